# Initial kernel scaffold; baseline (speedup 1.0000x reference)
import functools

import jax
import jax.numpy as jnp
from jax import lax
from jax.experimental import pallas as pl
from jax.experimental.pallas import tpu as pltpu
from jax.experimental.pallas import tpu_sc as plsc

B = 4096
S = 200
D = 32
C = 10
HALF = 100
NC = 2
NS = 16
NW = NC * NS
UB = (B * S) // (HALF * NW)
CB = B // NW
NBUF = 4

_mesh = plsc.VectorSubcoreMesh(core_axis_name="c", subcore_axis_name="s")


@functools.partial(
    pl.kernel,
    mesh=_mesh,
    out_type=jax.ShapeDtypeStruct((B, D), jnp.float32),
    scratch_types=[
        pltpu.VMEM((UB, HALF), jnp.int32),
        pltpu.VMEM((CB, D), jnp.float32),
    ]
    + [pltpu.VMEM((HALF, D), jnp.float32) for _ in range(NBUF)]
    + [pltpu.SemaphoreType.DMA for _ in range(NBUF)],
)
def _pool(table, x2, out, idx_v, pooled_v, b0, b1, b2, b3, s0, s1, s2, s3):
    bufs = (b0, b1, b2, b3)
    sems = (s0, s1, s2, s3)
    wid = lax.axis_index("s") * NC + lax.axis_index("c")
    ubase = wid * UB
    bbase = wid * CB

    pltpu.sync_copy(x2.at[pl.ds(ubase, UB)], idx_v)

    for b in range(NBUF):
        pltpu.async_copy(table.at[idx_v.at[b]], bufs[b], sems[b])

    def outer(g, _):
        u0 = 4 * g
        for jj in range(2):
            jrow = 2 * g + jj
            acc = (jnp.zeros((16,), jnp.float32),) * 4
            for b2 in range(2):
                b = 2 * jj + b2
                u = u0 + b
                pltpu.make_async_copy(
                    table.at[idx_v.at[u]], bufs[b], sems[b]).wait()
                buf = bufs[b]

                def red(i, a, buf=buf):
                    a0, a1, a2, a3 = a
                    r = i * 4
                    a0 = a0 + buf[r, pl.ds(0, 16)]
                    a1 = a1 + buf[r, pl.ds(16, 16)]
                    a2 = a2 + buf[r + 1, pl.ds(0, 16)]
                    a3 = a3 + buf[r + 1, pl.ds(16, 16)]
                    a0 = a0 + buf[r + 2, pl.ds(0, 16)]
                    a1 = a1 + buf[r + 2, pl.ds(16, 16)]
                    a2 = a2 + buf[r + 3, pl.ds(0, 16)]
                    a3 = a3 + buf[r + 3, pl.ds(16, 16)]
                    return (a0, a1, a2, a3)

                acc = lax.fori_loop(0, HALF // 4, red, acc)

                @pl.when(u + NBUF < UB)
                def _(b=b, u=u):
                    pltpu.async_copy(
                        table.at[idx_v.at[u + NBUF]], bufs[b], sems[b])

            pooled_v[jrow, pl.ds(0, 16)] = acc[0] + acc[2]
            pooled_v[jrow, pl.ds(16, 16)] = acc[1] + acc[3]
        return 0

    lax.fori_loop(0, CB // 2, outer, 0)
    pltpu.sync_copy(pooled_v, out.at[pl.ds(bbase, CB)])


def _linear_body(p_ref, w_ref, b_ref, o_ref):
    o_ref[...] = (
        jnp.dot(p_ref[...], w_ref[...], preferred_element_type=jnp.float32)
        + b_ref[...]
    )


def _linear(pooled, w, b):
    return pl.pallas_call(
        _linear_body,
        out_shape=jax.ShapeDtypeStruct((B, C), jnp.float32),
        grid=(4,),
        in_specs=[
            pl.BlockSpec((B // 4, D), lambda i: (i, 0)),
            pl.BlockSpec((D, C), lambda i: (0, 0)),
            pl.BlockSpec((1, C), lambda i: (0, 0)),
        ],
        out_specs=pl.BlockSpec((B // 4, C), lambda i: (i, 0)),
    )(pooled, w, b)


def kernel(x, emb_table, fc_w, fc_b):
    x2 = x.astype(jnp.int32).reshape(B * S // HALF, HALF)
    pooled = _pool(emb_table, x2)
    w = fc_w.T.astype(jnp.float32) * jnp.float32(1.0 / S)
    return _linear(pooled, w, fc_b.reshape(1, C).astype(jnp.float32))


# speedup vs baseline: 2.3353x; 2.3353x over previous
import functools

import jax
import jax.numpy as jnp
from jax import lax
from jax.experimental import pallas as pl
from jax.experimental.pallas import tpu as pltpu
from jax.experimental.pallas import tpu_sc as plsc

B = 4096
S = 200
D = 32
C = 10
HALF = 100
NC = 2
NS = 16
NW = NC * NS
UB = (B * S) // (HALF * NW)
CB = B // NW
NBUF = 4

_mesh = plsc.VectorSubcoreMesh(core_axis_name="c", subcore_axis_name="s")


@functools.partial(
    pl.kernel,
    mesh=_mesh,
    compiler_params=pltpu.CompilerParams(use_tc_tiling_on_sc=False),
    out_type=jax.ShapeDtypeStruct((B, D), jnp.float32),
    scratch_types=[
        pltpu.VMEM((UB, HALF), jnp.int32),
        pltpu.VMEM((CB, D), jnp.float32),
    ]
    + [pltpu.VMEM((HALF, D), jnp.float32) for _ in range(NBUF)]
    + [pltpu.SemaphoreType.DMA for _ in range(NBUF)],
)
def _pool(table, x2, out, idx_v, pooled_v, b0, b1, b2, b3, s0, s1, s2, s3):
    bufs = (b0, b1, b2, b3)
    sems = (s0, s1, s2, s3)
    wid = lax.axis_index("s") * NC + lax.axis_index("c")
    ubase = wid * UB
    bbase = wid * CB

    pltpu.sync_copy(x2.at[pl.ds(ubase, UB)], idx_v)

    for b in range(NBUF):
        pltpu.async_copy(table.at[idx_v.at[b]], bufs[b], sems[b])

    def outer(g, _):
        u0 = 4 * g
        for jj in range(2):
            jrow = 2 * g + jj
            acc = (jnp.zeros((16,), jnp.float32),) * 4
            for b2 in range(2):
                b = 2 * jj + b2
                u = u0 + b
                pltpu.make_async_copy(
                    table.at[idx_v.at[u]], bufs[b], sems[b]).wait()
                buf = bufs[b]

                def red(i, a, buf=buf):
                    a0, a1, a2, a3 = a
                    r = i * 4
                    a0 = a0 + buf[r, pl.ds(0, 16)]
                    a1 = a1 + buf[r, pl.ds(16, 16)]
                    a2 = a2 + buf[r + 1, pl.ds(0, 16)]
                    a3 = a3 + buf[r + 1, pl.ds(16, 16)]
                    a0 = a0 + buf[r + 2, pl.ds(0, 16)]
                    a1 = a1 + buf[r + 2, pl.ds(16, 16)]
                    a2 = a2 + buf[r + 3, pl.ds(0, 16)]
                    a3 = a3 + buf[r + 3, pl.ds(16, 16)]
                    return (a0, a1, a2, a3)

                acc = lax.fori_loop(0, HALF // 4, red, acc)

                @pl.when(u + NBUF < UB)
                def _(b=b, u=u):
                    pltpu.async_copy(
                        table.at[idx_v.at[u + NBUF]], bufs[b], sems[b])

            pooled_v[jrow, pl.ds(0, 16)] = acc[0] + acc[2]
            pooled_v[jrow, pl.ds(16, 16)] = acc[1] + acc[3]
        return 0

    lax.fori_loop(0, CB // 2, outer, 0)
    pltpu.sync_copy(pooled_v, out.at[pl.ds(bbase, CB)])


def _linear_body(p_ref, w_ref, b_ref, o_ref):
    o_ref[...] = (
        jnp.dot(p_ref[...], w_ref[...], preferred_element_type=jnp.float32)
        + b_ref[...]
    )


def _linear(pooled, w, b):
    return pl.pallas_call(
        _linear_body,
        out_shape=jax.ShapeDtypeStruct((B, C), jnp.float32),
        grid=(4,),
        in_specs=[
            pl.BlockSpec((B // 4, D), lambda i: (i, 0)),
            pl.BlockSpec((D, C), lambda i: (0, 0)),
            pl.BlockSpec((1, C), lambda i: (0, 0)),
        ],
        out_specs=pl.BlockSpec((B // 4, C), lambda i: (i, 0)),
    )(pooled, w, b)


def kernel(x, emb_table, fc_w, fc_b):
    x2 = x.astype(jnp.int32).reshape(B * S // HALF, HALF)
    pooled = _pool(emb_table, x2)
    w = fc_w.T.astype(jnp.float32) * jnp.float32(1.0 / S)
    return _linear(pooled, w, fc_b.reshape(1, C).astype(jnp.float32))


# speedup vs baseline: 2.4320x; 1.0414x over previous
import functools

import jax
import jax.numpy as jnp
from jax import lax
from jax.experimental import pallas as pl
from jax.experimental.pallas import tpu as pltpu
from jax.experimental.pallas import tpu_sc as plsc

B = 4096
S = 200
D = 32
C = 10
HALF = 100
NC = 2
NS = 16
NW = NC * NS
UB = (B * S) // (HALF * NW)
CB = B // NW
NBUF = 8

_mesh = plsc.VectorSubcoreMesh(core_axis_name="c", subcore_axis_name="s")


@functools.partial(
    pl.kernel,
    mesh=_mesh,
    compiler_params=pltpu.CompilerParams(use_tc_tiling_on_sc=False),
    out_type=jax.ShapeDtypeStruct((B, D), jnp.float32),
    scratch_types=[
        pltpu.VMEM((UB, HALF), jnp.int32),
        pltpu.VMEM((CB, D), jnp.float32),
    ]
    + [pltpu.VMEM((HALF, D), jnp.float32) for _ in range(NBUF)]
    + [pltpu.SemaphoreType.DMA for _ in range(NBUF)],
)
def _pool(table, x2, out, idx_v, pooled_v,
          b0, b1, b2, b3, b4, b5, b6, b7,
          s0, s1, s2, s3, s4, s5, s6, s7):
    bufs = (b0, b1, b2, b3, b4, b5, b6, b7)
    sems = (s0, s1, s2, s3, s4, s5, s6, s7)
    wid = lax.axis_index("s") * NC + lax.axis_index("c")
    ubase = wid * UB
    bbase = wid * CB

    pltpu.sync_copy(x2.at[pl.ds(ubase, UB)], idx_v)

    for b in range(NBUF):
        pltpu.async_copy(table.at[idx_v.at[b]], bufs[b], sems[b])

    def outer(g, _):
        u0 = NBUF * g
        for jj in range(NBUF // 2):
            jrow = (NBUF // 2) * g + jj
            acc = (jnp.zeros((16,), jnp.float32),) * 4
            for b2 in range(2):
                b = 2 * jj + b2
                u = u0 + b
                pltpu.make_async_copy(
                    table.at[idx_v.at[u]], bufs[b], sems[b]).wait()
                buf = bufs[b]

                def red(i, a, buf=buf):
                    a0, a1, a2, a3 = a
                    r = i * 10
                    for q in range(0, 10, 2):
                        a0 = a0 + buf[r + q, pl.ds(0, 16)]
                        a1 = a1 + buf[r + q, pl.ds(16, 16)]
                        a2 = a2 + buf[r + q + 1, pl.ds(0, 16)]
                        a3 = a3 + buf[r + q + 1, pl.ds(16, 16)]
                    return (a0, a1, a2, a3)

                acc = lax.fori_loop(0, HALF // 10, red, acc)

                @pl.when(u + NBUF < UB)
                def _(b=b, u=u):
                    pltpu.async_copy(
                        table.at[idx_v.at[u + NBUF]], bufs[b], sems[b])

            pooled_v[jrow, pl.ds(0, 16)] = acc[0] + acc[2]
            pooled_v[jrow, pl.ds(16, 16)] = acc[1] + acc[3]
        return 0

    lax.fori_loop(0, UB // NBUF, outer, 0)
    pltpu.sync_copy(pooled_v, out.at[pl.ds(bbase, CB)])


def _linear_body(p_ref, w_ref, b_ref, o_ref):
    o_ref[...] = (
        jnp.dot(p_ref[...], w_ref[...], preferred_element_type=jnp.float32)
        + b_ref[...]
    )


def _linear(pooled, w, b):
    return pl.pallas_call(
        _linear_body,
        out_shape=jax.ShapeDtypeStruct((B, C), jnp.float32),
        grid=(4,),
        in_specs=[
            pl.BlockSpec((B // 4, D), lambda i: (i, 0)),
            pl.BlockSpec((D, C), lambda i: (0, 0)),
            pl.BlockSpec((1, C), lambda i: (0, 0)),
        ],
        out_specs=pl.BlockSpec((B // 4, C), lambda i: (i, 0)),
    )(pooled, w, b)


def kernel(x, emb_table, fc_w, fc_b):
    x2 = x.astype(jnp.int32).reshape(B * S // HALF, HALF)
    pooled = _pool(emb_table, x2)
    w = fc_w.T.astype(jnp.float32) * jnp.float32(1.0 / S)
    return _linear(pooled, w, fc_b.reshape(1, C).astype(jnp.float32))


# speedup vs baseline: 2.4388x; 1.0028x over previous
import functools

import jax
import jax.numpy as jnp
from jax import lax
from jax.experimental import pallas as pl
from jax.experimental.pallas import tpu as pltpu
from jax.experimental.pallas import tpu_sc as plsc

B = 4096
S = 200
D = 32
C = 10
UA = 104
UBN = 96
NC = 2
NS = 16
NW = NC * NS
CB = B // NW
UB = CB * 2
NBUF = 8

_mesh = plsc.VectorSubcoreMesh(core_axis_name="c", subcore_axis_name="s")


@functools.partial(
    pl.kernel,
    mesh=_mesh,
    compiler_params=pltpu.CompilerParams(use_tc_tiling_on_sc=False),
    out_type=jax.ShapeDtypeStruct((B, D), jnp.float32),
    scratch_types=[
        pltpu.VMEM((CB, S), jnp.int32),
        pltpu.VMEM((CB, D), jnp.float32),
    ]
    + [pltpu.VMEM((UA, D), jnp.float32) for _ in range(NBUF)]
    + [pltpu.SemaphoreType.DMA for _ in range(NBUF)],
)
def _pool(table, x2, out, idx_v, pooled_v,
          b0, b1, b2, b3, b4, b5, b6, b7,
          s0, s1, s2, s3, s4, s5, s6, s7):
    bufs = (b0, b1, b2, b3, b4, b5, b6, b7)
    sems = (s0, s1, s2, s3, s4, s5, s6, s7)
    wid = lax.axis_index("s") * NC + lax.axis_index("c")
    bbase = wid * CB

    pltpu.sync_copy(x2.at[pl.ds(bbase, CB)], idx_v)

    def _unit(u, b):
        j = u // 2
        if b % 2 == 0:
            return idx_v.at[j, pl.ds(0, UA)], bufs[b]
        return idx_v.at[j, pl.ds(UA, UBN)], bufs[b].at[pl.ds(0, UBN)]

    def _start(u, b):
        src, dst = _unit(u, b)
        pltpu.async_copy(table.at[src], dst, sems[b])

    for b in range(NBUF):
        _start(b, b)

    def outer(g, _):
        u0 = NBUF * g
        for jj in range(NBUF // 2):
            jrow = (NBUF // 2) * g + jj
            acc = (jnp.zeros((16,), jnp.float32),) * 4
            for b2 in range(2):
                b = 2 * jj + b2
                u = u0 + b
                src, dst = _unit(u, b)
                pltpu.make_async_copy(table.at[src], dst, sems[b]).wait()
                buf = bufs[b]
                nrows = UA if b % 2 == 0 else UBN

                def red(i, a, buf=buf):
                    a0, a1, a2, a3 = a
                    r = i * 8
                    for q in range(0, 8, 2):
                        a0 = a0 + buf[r + q, pl.ds(0, 16)]
                        a1 = a1 + buf[r + q, pl.ds(16, 16)]
                        a2 = a2 + buf[r + q + 1, pl.ds(0, 16)]
                        a3 = a3 + buf[r + q + 1, pl.ds(16, 16)]
                    return (a0, a1, a2, a3)

                acc = lax.fori_loop(0, nrows // 8, red, acc)

                @pl.when(u + NBUF < UB)
                def _(b=b, u=u):
                    _start(u + NBUF, b)

            pooled_v[jrow, pl.ds(0, 16)] = acc[0] + acc[2]
            pooled_v[jrow, pl.ds(16, 16)] = acc[1] + acc[3]
        return 0

    lax.fori_loop(0, UB // NBUF, outer, 0)
    pltpu.sync_copy(pooled_v, out.at[pl.ds(bbase, CB)])


def _linear_body(p_ref, w_ref, b_ref, o_ref):
    o_ref[...] = (
        jnp.dot(p_ref[...], w_ref[...], preferred_element_type=jnp.float32)
        + b_ref[...]
    )


def _linear(pooled, w, b):
    return pl.pallas_call(
        _linear_body,
        out_shape=jax.ShapeDtypeStruct((B, C), jnp.float32),
        grid=(4,),
        in_specs=[
            pl.BlockSpec((B // 4, D), lambda i: (i, 0)),
            pl.BlockSpec((D, C), lambda i: (0, 0)),
            pl.BlockSpec((1, C), lambda i: (0, 0)),
        ],
        out_specs=pl.BlockSpec((B // 4, C), lambda i: (i, 0)),
    )(pooled, w, b)


def kernel(x, emb_table, fc_w, fc_b):
    pooled = _pool(emb_table, x.astype(jnp.int32))
    w = fc_w.T.astype(jnp.float32) * jnp.float32(1.0 / S)
    return _linear(pooled, w, fc_b.reshape(1, C).astype(jnp.float32))
